# baseline (speedup 1.0000x reference)
import functools

import jax
import jax.numpy as jnp
from jax import lax
from jax.experimental import pallas as pl
from jax.experimental.pallas import tpu as pltpu
from jax.experimental.pallas import tpu_sc as plsc

N = 50000
E = 800000
IN_DIM = 128
HID = 256
OUT = 128
NB = 128
BK = 64
NC = 2
NH = 4

ROWS = 400
NSUB = 16
EPT = E // NSUB
C = 400
NCHUNK = EPT // C
QPAD = 8448
NQ = 6
ACC_R = QPAD + 8
WR = QPAD // NSUB
ZR = WR + 8


def _make_agg_kernel(ncb):
    mesh = plsc.VectorSubcoreMesh(core_axis_name="c", subcore_axis_name="s")
    npass = ncb * NQ // 2

    def body(y_ref, src_ref, dst_ref, zero_ref, out_ref,
             src_v, dst_v, gidx_v, lidx_v, rows_v, acc, sem):
        c = lax.axis_index("c")
        s = lax.axis_index("s")
        dump = jnp.arange(16, dtype=jnp.int32) % 8 + QPAD
        for p in range(npass):
            k = 2 * p + c
            cb = k // NQ
            q0 = (k % NQ) * QPAD

            pltpu.sync_copy(zero_ref, acc.at[pl.ds(s * WR, ZR)])
            plsc.subcore_barrier()

            @pl.loop(0, NCHUNK)
            def _chunk(i):
                off = s * EPT + i * C
                pltpu.sync_copy(src_ref.at[pl.ds(off, C)], src_v)
                pltpu.sync_copy(dst_ref.at[pl.ds(off, C)], dst_v)

                @pl.loop(0, C // 16)
                def _gi(j):
                    sv = src_v[pl.ds(j * 16, 16)]
                    gidx_v[pl.ds(j * 16, 16)] = sv * ncb + cb
                    dv = dst_v[pl.ds(j * 16, 16)] - q0
                    ok = (dv >= 0) & (dv < QPAD)
                    lidx_v[pl.ds(j * 16, 16)] = jnp.where(ok, dv, dump)

                pltpu.async_copy(y_ref.at[gidx_v], rows_v, sem).wait()
                pltpu.sync_copy(rows_v, acc.at[lidx_v], add=True)

            plsc.subcore_barrier()

            pltpu.sync_copy(
                acc.at[pl.ds(s * WR, WR)],
                out_ref.at[pl.ds(k * QPAD + s * WR, WR)])
            plsc.subcore_barrier()

    return pl.kernel(
        body,
        out_type=jax.ShapeDtypeStruct((ncb * NQ * QPAD, 128), jnp.float32),
        mesh=mesh,
        scratch_types=[
            pltpu.VMEM((C,), jnp.int32),
            pltpu.VMEM((C,), jnp.int32),
            pltpu.VMEM((C,), jnp.int32),
            pltpu.VMEM((C,), jnp.int32),
            pltpu.VMEM((C, 128), jnp.float32),
            pltpu.VMEM_SHARED((ACC_R, 128), jnp.float32),
            pltpu.SemaphoreType.DMA,
        ],
    )


_agg2_kernel = _make_agg_kernel(2)
_agg1_kernel = _make_agg_kernel(1)


def _gelu(x):
    return 0.5 * x * (1.0 + jax.lax.erf(x * 0.7071067811865476))


def _l0_body(x_ref, w_ref, no_ref, y_ref):
    h = jnp.dot(x_ref[...], w_ref[...], preferred_element_type=jnp.float32)
    y_ref[...] = h * no_ref[...]


def _layer0(feat, W0, norm_out):
    return pl.pallas_call(
        _l0_body,
        grid=(N // ROWS,),
        in_specs=[
            pl.BlockSpec((ROWS, IN_DIM), lambda i: (i, 0)),
            pl.BlockSpec((IN_DIM, HID), lambda i: (0, 0)),
            pl.BlockSpec((ROWS, 1), lambda i: (i, 0)),
        ],
        out_specs=pl.BlockSpec((ROWS, HID), lambda i: (i, 0)),
        out_shape=jax.ShapeDtypeStruct((N, HID), jnp.float32),
    )(feat, W0, norm_out)


def _mid_body(nblk, a_ref, ni_ref, b_ref, w_ref, no_ref, y_ref):
    x = jnp.concatenate([a_ref[b] for b in range(nblk)], axis=-1)
    x = x * ni_ref[...] + b_ref[...]
    x = _gelu(x)
    h = jnp.dot(x, w_ref[...], preferred_element_type=jnp.float32)
    y_ref[...] = h * no_ref[...]


def _mid_layer(a, norm_in, b_prev, W, norm_out, din, dout):
    nblk = din // 128
    return pl.pallas_call(
        functools.partial(_mid_body, nblk),
        grid=(N // ROWS,),
        in_specs=[
            pl.BlockSpec((nblk, ROWS, 128), lambda i: (0, i, 0)),
            pl.BlockSpec((ROWS, 1), lambda i: (i, 0)),
            pl.BlockSpec((din,), lambda i: (0,)),
            pl.BlockSpec((din, dout), lambda i: (0, 0)),
            pl.BlockSpec((ROWS, 1), lambda i: (i, 0)),
        ],
        out_specs=pl.BlockSpec((ROWS, dout), lambda i: (i, 0)),
        out_shape=jax.ShapeDtypeStruct((N, dout), jnp.float32),
    )(a, norm_in, b_prev, W, norm_out)


def _attn_body(a_ref, ni_ref, b3_ref, w1_ref, b1_ref, w2_ref,
               ln_g_ref, ln_b_ref, hn_ref, s_ref):
    x = a_ref[0]
    x = x * ni_ref[...] + b3_ref[...]
    m = jnp.mean(x, axis=-1, keepdims=True)
    v = jnp.mean((x - m) ** 2, axis=-1, keepdims=True)
    hn = (x - m) * jax.lax.rsqrt(v + 1e-5) * ln_g_ref[...] + ln_b_ref[...]
    hn_ref[...] = hn
    hh = jnp.dot(hn, w1_ref[...], preferred_element_type=jnp.float32) + b1_ref[...]
    hh = _gelu(hh)
    s_ref[...] = jnp.dot(hh, w2_ref[...], preferred_element_type=jnp.float32)


def _attn_scores(a3, norm_in, b3, ln_g, ln_b, Ha1, ba1, Ha2, ba2):
    w1 = jnp.transpose(Ha1, (1, 0, 2)).reshape(OUT, NH * 128)
    b1 = ba1.reshape(NH * 128)
    w2col = (jnp.transpose(Ha2, (0, 2, 1)).reshape(NH * 128) / NH)
    w2 = jnp.zeros((NH * 128, 128), jnp.float32).at[:, 0].set(w2col)
    hn, s = pl.pallas_call(
        _attn_body,
        grid=(N // ROWS,),
        in_specs=[
            pl.BlockSpec((1, ROWS, 128), lambda i: (0, i, 0)),
            pl.BlockSpec((ROWS, 1), lambda i: (i, 0)),
            pl.BlockSpec((OUT,), lambda i: (0,)),
            pl.BlockSpec((OUT, NH * 128), lambda i: (0, 0)),
            pl.BlockSpec((NH * 128,), lambda i: (0,)),
            pl.BlockSpec((NH * 128, 128), lambda i: (0, 0)),
            pl.BlockSpec((OUT,), lambda i: (0,)),
            pl.BlockSpec((OUT,), lambda i: (0,)),
        ],
        out_specs=[
            pl.BlockSpec((ROWS, OUT), lambda i: (i, 0)),
            pl.BlockSpec((ROWS, 128), lambda i: (i, 0)),
        ],
        out_shape=[
            jax.ShapeDtypeStruct((N, OUT), jnp.float32),
            jax.ShapeDtypeStruct((N, 128), jnp.float32),
        ],
    )(a3, norm_in, b3, w1, b1, w2, ln_g, ln_b)
    scores = s[:, :1] + jnp.mean(ba2, axis=0)
    return hn, scores


def kernel(feat, edge_index, bag_indices, W0, b0, W1, b1, W2, b2, W3, b3,
           ln_g, ln_b, Ha1, ba1, Ha2, ba2, Wc1, bc1, lnc_g, lnc_b, Wc2, bc2):
    src = edge_index[0]
    dst = edge_index[1]
    zero = jnp.zeros((ZR, 128), jnp.float32)

    def agg(y, din, s_idx, d_idx):
        ncb = din // 128
        k = _agg2_kernel if ncb == 2 else _agg1_kernel
        out = k(y.reshape(N * ncb, 128), s_idx, d_idx, zero)
        return out.reshape(ncb, NQ * QPAD, 128)[:, :N]

    ones_tab = jnp.ones((N, 128), jnp.float32)
    deg_in = agg(ones_tab, 128, src, dst)[0, :, :1]
    deg_out = agg(ones_tab, 128, dst, src)[0, :, :1]
    norm_out = jnp.clip(deg_out, 1.0, None) ** -0.5
    norm_in = jnp.clip(deg_in, 1.0, None) ** -0.5

    y0 = _layer0(feat, W0, norm_out)
    a0 = agg(y0, HID, src, dst)
    y1 = _mid_layer(a0, norm_in, b0, W1, norm_out, HID, HID)
    a1 = agg(y1, HID, src, dst)
    y2 = _mid_layer(a1, norm_in, b1, W2, norm_out, HID, HID)
    a2 = agg(y2, HID, src, dst)
    y3 = _mid_layer(a2, norm_in, b2, W3, norm_out, HID, OUT)
    a3 = agg(y3, OUT, src, dst)

    h, scores = _attn_scores(a3, norm_in, b3, ln_g, ln_b, Ha1, ba1, Ha2, ba2)

    weights = jax.nn.softmax(scores, axis=0)
    wb = weights[bag_indices]
    hb = h[bag_indices]
    bag_feats = jnp.sum(wb * hb, axis=1)
    z = bag_feats @ Wc1 + bc1
    zm = jnp.mean(z, axis=-1, keepdims=True)
    zv = jnp.var(z, axis=-1, keepdims=True)
    z = (z - zm) / jnp.sqrt(zv + 1e-5) * lnc_g + lnc_b
    z = _gelu(z)
    logits = z @ Wc2 + bc2
    return logits
